# baseline (speedup 1.0000x reference)
import functools

import numpy as np
import jax
import jax.numpy as jnp
from jax import lax
from jax.experimental import pallas as pl
from jax.experimental.pallas import tpu as pltpu
from jax.experimental.pallas import tpu_sc as plsc

NUM_ADDRESSES = 100000
ADDRESS_DIM = 512
CONTENT_DIM = 512
BATCH = 256
THRESHOLD = 76

BLK = 4000
WPB = 128
NBLK = NUM_ADDRESSES // BLK
WORDS_PER_ROW = NBLK * WPB
NW = 32
ROWS_PER_TILE = BATCH // NW
TILE_WORDS = ROWS_PER_TILE * WORDS_PER_ROW

CAPW = 1024
CAPP = 1024
CHUNK = 32
DGRP = CONTENT_DIM // 16


def _pack_weights():
    c = np.arange(BLK)
    g = c // 32
    bit = c % 32
    glo = np.zeros((BLK, WPB), np.float32)
    ghi = np.zeros((BLK, WPB), np.float32)
    lo = bit < 16
    glo[c[lo], g[lo]] = (2.0 ** bit[lo])
    ghi[c[~lo], g[~lo]] = (2.0 ** (bit[~lo] - 16))
    return glo, ghi


_GLO_NP, _GHI_NP = _pack_weights()


def _tc_pack_body(address_ref, addresses_ref, glo_ref, ghi_ref, out_ref):
    sim = lax.dot_general(
        address_ref[...].astype(jnp.bfloat16),
        addresses_ref[...].astype(jnp.bfloat16),
        (((1,), (1,)), ((), ())),
        preferred_element_type=jnp.float32,
    )
    mask = (sim >= THRESHOLD).astype(jnp.bfloat16)
    lo = lax.dot_general(mask, glo_ref[...], (((1,), (0,)), ((), ())),
                         preferred_element_type=jnp.float32)
    hi = lax.dot_general(mask, ghi_ref[...], (((1,), (0,)), ((), ())),
                         preferred_element_type=jnp.float32)
    out_ref[...] = lo.astype(jnp.int32) | (hi.astype(jnp.int32) << 16)


def _tc_pack(address, addresses, glo, ghi):
    return pl.pallas_call(
        _tc_pack_body,
        grid=(NBLK,),
        in_specs=[
            pl.BlockSpec((BATCH, ADDRESS_DIM), lambda j: (0, 0)),
            pl.BlockSpec((BLK, ADDRESS_DIM), lambda j: (j, 0)),
            pl.BlockSpec((BLK, WPB), lambda j: (0, 0)),
            pl.BlockSpec((BLK, WPB), lambda j: (0, 0)),
        ],
        out_specs=pl.BlockSpec((BATCH, WPB), lambda j: (0, j)),
        out_shape=jax.ShapeDtypeStruct((BATCH, WORDS_PER_ROW), jnp.int32),
    )(address, addresses, glo, ghi)


def _sc_body(l1_hbm, content_hbm, out_hbm,
             maskbuf, wbuf, colbuf, idxbuf, gbuf, acc, sem):
    wid = lax.axis_index("s") * 2 + lax.axis_index("c")
    row0 = wid * ROWS_PER_TILE
    lanes = lax.iota(jnp.int32, 16)
    zeros16 = jnp.zeros((16,), jnp.int32)
    ones16 = jnp.ones((16,), jnp.int32)

    pltpu.sync_copy(l1_hbm.at[pl.ds(wid * TILE_WORDS, TILE_WORDS)], maskbuf)

    def row_body(r, carry):
        rowbase = r * WORDS_PER_ROW

        def scan_body(i, wcnt):
            v = maskbuf[pl.ds(rowbase + i * 16, 16)]
            nz = v != 0
            nzi = jnp.where(nz, ones16, zeros16)
            cnt = jnp.sum(nzi)

            ok = wcnt <= CAPW - 16

            @pl.when(jnp.logical_and(cnt > 0, ok))
            def _():
                pos = wcnt + plsc.cumsum(nzi) - nzi
                plsc.store_scatter(wbuf, [pos], i * 16 + lanes, mask=nz)

            return wcnt + jnp.where(ok, cnt, 0)

        wcnt = lax.fori_loop(0, WORDS_PER_ROW // 16, scan_body, 0)
        wcnt = jnp.minimum(wcnt, CAPW)
        plsc.subcore_barrier()

        def dec_body(j, pcnt):
            raw = wbuf[pl.ds(j * 16, 16)]
            in_range = (j * 16 + lanes) < wcnt
            wv = jnp.where(in_range, raw, WORDS_PER_ROW - 1)
            vals = plsc.load_gather(maskbuf, [rowbase + wv])
            vals = jnp.where(in_range, vals, zeros16)
            colbase = BLK * (wv >> 7) + 32 * (wv & 127)

            t = vals - ((vals >> 1) & 0x55555555)
            t = (t & 0x33333333) + ((t >> 2) & 0x33333333)
            t = (t + (t >> 4)) & 0x0F0F0F0F
            nbits = (t * 0x01010101) >> 24

            offs = pcnt + plsc.cumsum(nbits) - nbits
            guard = pcnt <= CAPP - 512

            @pl.when(guard)
            def _():
                rc = offs
                for b in range(32):
                    bit = (vals >> b) & 1
                    plsc.store_scatter(colbuf, [rc], colbase + b,
                                       mask=bit == 1)
                    rc = rc + bit

            return pcnt + jnp.where(guard, jnp.sum(nbits), 0)

        pcnt = lax.fori_loop(0, (wcnt + 15) // 16, dec_body, 0)
        plsc.subcore_barrier()

        for d in range(DGRP):
            acc[r, pl.ds(d * 16, 16)] = jnp.zeros((16,), jnp.float32)

        nch = (pcnt + CHUNK - 1) // CHUNK

        def ch_body(ch, c2):
            base = ch * CHUNK
            for q in range(CHUNK // 16):
                idx = base + q * 16 + lanes
                live = idx < pcnt
                cc = plsc.load_gather(colbuf, [idx])
                idxbuf[pl.ds(q * 16, 16)] = jnp.where(live, cc, zeros16)
            pltpu.async_copy(content_hbm.at[idxbuf], gbuf, sem).wait()

            clen = jnp.minimum(CHUNK, pcnt - base)

            def acc_body(i, c3):
                for d in range(DGRP):
                    v = gbuf[i, pl.ds(d * 16, 16)]
                    b = plsc.bitcast(v, jnp.int32)
                    b = (b + 0x7FFF + ((b >> 16) & 1)) & jnp.int32(-65536)
                    plsc.addupdate(acc.at[r, pl.ds(d * 16, 16)],
                                   plsc.bitcast(b, jnp.float32))
                return c3

            lax.fori_loop(0, clen, acc_body, 0)
            return c2

        lax.fori_loop(0, nch, ch_body, 0)

        for d in range(DGRP):
            acc[r, pl.ds(d * 16, 16)] = jnp.sign(acc[r, pl.ds(d * 16, 16)])
        return carry

    lax.fori_loop(0, ROWS_PER_TILE, row_body, 0)

    pltpu.sync_copy(acc, out_hbm.at[pl.ds(row0, ROWS_PER_TILE)])


@functools.lru_cache(maxsize=1)
def _make_sc_read():
    @functools.partial(
        pl.kernel,
        out_type=jax.ShapeDtypeStruct((BATCH, CONTENT_DIM), jnp.float32),
        mesh=plsc.VectorSubcoreMesh(core_axis_name="c", subcore_axis_name="s"),
        compiler_params=pltpu.CompilerParams(needs_layout_passes=False),
        scratch_types=[
            pltpu.VMEM((TILE_WORDS,), jnp.int32),
            pltpu.VMEM((CAPW,), jnp.int32),
            pltpu.VMEM((CAPP,), jnp.int32),
            pltpu.VMEM((CHUNK,), jnp.int32),
            pltpu.VMEM((CHUNK, CONTENT_DIM), jnp.float32),
            pltpu.VMEM((ROWS_PER_TILE, CONTENT_DIM), jnp.float32),
            pltpu.SemaphoreType.DMA,
        ],
    )
    def _sc_read(l1_flat, content, out, *scratch):
        _sc_body(l1_flat, content, out, *scratch)

    return _sc_read


@jax.jit
def kernel(address, addresses, content):
    glo = jnp.asarray(_GLO_NP).astype(jnp.bfloat16)
    ghi = jnp.asarray(_GHI_NP).astype(jnp.bfloat16)
    l1 = _tc_pack(address, addresses, glo, ghi)
    return _make_sc_read()(l1.reshape(-1), content)
